# baseline (speedup 1.0000x reference)
import functools

import jax
import jax.numpy as jnp
from jax import lax
from jax.experimental import pallas as pl
from jax.experimental.pallas import tpu as pltpu
from jax.experimental.pallas import tpu_sc as plsc

_ROWS = 128
_COLS = 32768
_K = 64


@functools.cache
def _make_gather():
    info = plsc.get_sparse_core_info()
    nc, ns, lanes = info.num_cores, info.num_subcores, info.num_lanes
    nw = nc * ns
    rows_per_w = _ROWS // nw
    elems_per_w = rows_per_w * _K
    n_chunks = elems_per_w // 128

    mesh = plsc.VectorSubcoreMesh(core_axis_name="c", subcore_axis_name="s")

    @functools.partial(
        pl.kernel,
        mesh=mesh,
        out_type=jax.ShapeDtypeStruct((_ROWS * _K,), jnp.float32),
        scratch_types=[
            pltpu.VMEM((_K,), jnp.int32),
            pltpu.VMEM((n_chunks, 128), jnp.int32),
            pltpu.VMEM((elems_per_w,), jnp.float32),
            pltpu.SemaphoreType.DMA,
        ],
    )
    def gather_kernel(x_hbm, mask_hbm, out_hbm, mask_v, idx_v, vals_v, sem):
        wid = lax.axis_index("s") * nc + lax.axis_index("c")
        pltpu.sync_copy(mask_hbm, mask_v)
        row0 = wid * rows_per_w
        for p0 in range(0, elems_per_w, lanes):
            r = p0 // _K
            j0 = p0 % _K
            base = (row0 + r) * _COLS
            idx_v[p0 // 128, pl.ds(p0 % 128, lanes)] = mask_v[pl.ds(j0, lanes)] + base
        copies = [
            pltpu.async_copy(
                x_hbm.at[idx_v.at[c]], vals_v.at[pl.ds(c * 128, 128)], sem
            )
            for c in range(n_chunks)
        ]
        for cp in copies:
            cp.wait()
        pltpu.sync_copy(vals_v, out_hbm.at[pl.ds(wid * elems_per_w, elems_per_w)])

    return gather_kernel


def kernel(x, mask):
    out_flat = _make_gather()(x.reshape(_ROWS * _COLS), mask)
    return out_flat.reshape(_ROWS, _K)


# speedup vs baseline: 1.4320x; 1.4320x over previous
import functools

import jax
import jax.numpy as jnp
from jax import lax
from jax.experimental import pallas as pl
from jax.experimental.pallas import tpu as pltpu
from jax.experimental.pallas import tpu_sc as plsc

_ROWS = 128
_COLS = 32768
_K = 64
_SUB = 8
_LANE = 128


@functools.cache
def _make_gather():
    info = plsc.get_sparse_core_info()
    nc, ns = info.num_cores, info.num_subcores
    n_blocks = _ROWS // _SUB

    mesh = plsc.VectorSubcoreMesh(core_axis_name="c", subcore_axis_name="s")

    @functools.partial(
        pl.kernel,
        mesh=mesh,
        out_type=jax.ShapeDtypeStruct((_ROWS, _K), jnp.float32),
        scratch_types=[
            pltpu.VMEM((_K,), jnp.int32),
            pltpu.VMEM((_K * _SUB, _LANE), jnp.float32),
            pltpu.VMEM((_SUB, _K), jnp.float32),
            pltpu.SemaphoreType.DMA,
        ],
        compiler_params=pltpu.CompilerParams(needs_layout_passes=False),
    )
    def gather_kernel(
        x_hbm, mask_hbm, out_hbm, mask_v, blocks_v, vals_v, sem
    ):
        wid = lax.axis_index("s") * nc + lax.axis_index("c")

        @pl.when(wid < n_blocks)
        def _():
            pltpu.sync_copy(mask_hbm, mask_v)
            r0 = pl.multiple_of(wid * _SUB, _SUB)
            mchunk = [mask_v[pl.ds(16 * c, 16)] for c in range(_K // 16)]
            msca = [mchunk[j // 16][j % 16] for j in range(_K)]
            copies = []
            for j in range(_K):
                mt = pl.multiple_of(
                    lax.shift_left(
                        lax.shift_right_logical(msca[j], 7), 7
                    ),
                    _LANE,
                )
                copies.append(
                    pltpu.async_copy(
                        x_hbm.at[pl.ds(r0, _SUB), pl.ds(mt, _LANE)],
                        blocks_v.at[pl.ds(j * _SUB, _SUB)],
                        sem,
                    )
                )
            for cp in copies:
                cp.wait()
            iota = lax.iota(jnp.int32, 16)
            row = lax.bitwise_and(iota, 7)
            half = lax.shift_right_logical(iota, 3)
            for p in range(_K // 2):
                l0 = lax.bitwise_and(msca[2 * p], 127)
                l1 = lax.bitwise_and(msca[2 * p + 1], 127)
                lane = jnp.where(iota < 8, l0, l1)
                col = half + (2 * p)
                vec = plsc.load_gather(blocks_v, [col * _SUB + row, lane])
                plsc.store_scatter(vals_v, [row, col], vec)
            pltpu.sync_copy(vals_v, out_hbm.at[pl.ds(r0, _SUB), :])

    return gather_kernel


def kernel(x, mask):
    return _make_gather()(x, mask)


# speedup vs baseline: 1.4960x; 1.0447x over previous
import functools

import jax
import jax.numpy as jnp
from jax import lax
from jax.experimental import pallas as pl
from jax.experimental.pallas import tpu as pltpu
from jax.experimental.pallas import tpu_sc as plsc

_ROWS = 128
_COLS = 32768
_K = 64
_SUB = 8
_LANE = 128


@functools.cache
def _make_gather():
    info = plsc.get_sparse_core_info()
    nc, ns = info.num_cores, info.num_subcores
    n_blocks = _ROWS // _SUB

    mesh = plsc.VectorSubcoreMesh(core_axis_name="c", subcore_axis_name="s")

    @functools.partial(
        pl.kernel,
        mesh=mesh,
        out_type=jax.ShapeDtypeStruct((_ROWS, _K), jnp.float32),
        scratch_types=[
            pltpu.VMEM((_K,), jnp.int32),
            pltpu.VMEM((_K * _SUB, _LANE), jnp.float32),
            pltpu.VMEM((_SUB, _K), jnp.float32),
            pltpu.SemaphoreType.DMA,
        ],
        compiler_params=pltpu.CompilerParams(needs_layout_passes=False),
    )
    def gather_kernel(
        x_hbm, mask_hbm, out_hbm, mask_v, blocks_v, vals_v, sem
    ):
        wid = lax.axis_index("s") * nc + lax.axis_index("c")

        @pl.when(wid < n_blocks)
        def _():
            pltpu.sync_copy(mask_hbm, mask_v)
            r0 = pl.multiple_of(wid * _SUB, _SUB)
            iota = lax.iota(jnp.int32, 16)
            row = lax.bitwise_and(iota, 7)
            half = lax.shift_right_logical(iota, 3)

            def mask_scalar(j):
                base = pl.multiple_of(
                    lax.shift_left(lax.shift_right_logical(j, 4), 4), 16
                )
                chunk = mask_v[pl.ds(base, 16)]
                sel = jnp.where(iota == lax.bitwise_and(j, 15), chunk, 0)
                return jnp.sum(sel)

            def issue(j, carry):
                m = mask_scalar(j)
                mt = pl.multiple_of(
                    lax.shift_left(lax.shift_right_logical(m, 7), 7), _LANE
                )
                dst = pl.multiple_of(j * _SUB, _SUB)
                pltpu.async_copy(
                    x_hbm.at[pl.ds(r0, _SUB), pl.ds(mt, _LANE)],
                    blocks_v.at[pl.ds(dst, _SUB)],
                    sem,
                )
                return carry

            lax.fori_loop(0, _K, issue, 0)

            for q in range(4):
                pltpu.make_async_copy(
                    x_hbm.at[:, pl.ds(0, _LANE)],
                    blocks_v.at[pl.ds(q * _ROWS, _ROWS)],
                    sem,
                ).wait()

            def extract(p, carry):
                j0 = 2 * p
                l0 = lax.bitwise_and(mask_scalar(j0), 127)
                l1 = lax.bitwise_and(mask_scalar(j0 + 1), 127)
                lane = jnp.where(iota < 8, l0, l1)
                col = half + j0
                vec = plsc.load_gather(blocks_v, [col * _SUB + row, lane])
                plsc.store_scatter(vals_v, [row, col], vec)
                return carry

            lax.fori_loop(0, _K // 2, extract, 0)
            pltpu.sync_copy(vals_v, out_hbm.at[pl.ds(r0, _SUB), :])

    return gather_kernel


def kernel(x, mask):
    return _make_gather()(x, mask)


# speedup vs baseline: 1.5284x; 1.0216x over previous
import functools

import jax
import jax.numpy as jnp
from jax import lax
from jax.experimental import pallas as pl
from jax.experimental.pallas import tpu as pltpu
from jax.experimental.pallas import tpu_sc as plsc

_ROWS = 128
_COLS = 32768
_K = 64
_SUB = 8
_LANE = 128


@functools.cache
def _make_gather():
    info = plsc.get_sparse_core_info()
    nc, ns = info.num_cores, info.num_subcores
    n_blocks = _ROWS // _SUB

    mesh = plsc.VectorSubcoreMesh(core_axis_name="c", subcore_axis_name="s")

    @functools.partial(
        pl.kernel,
        mesh=mesh,
        out_type=jax.ShapeDtypeStruct((_ROWS, _K), jnp.float32),
        scratch_types=[
            pltpu.VMEM((_K,), jnp.int32),
            pltpu.VMEM((_K,), jnp.int32),
            pltpu.VMEM((_K * _SUB, _LANE), jnp.float32),
            pltpu.VMEM((_SUB, _K), jnp.float32),
            pltpu.SemaphoreType.DMA,
        ],
        compiler_params=pltpu.CompilerParams(needs_layout_passes=False),
    )
    def gather_kernel(
        x_hbm, mask_hbm, out_hbm, mask_v, lanes_v, blocks_v, vals_v, sem
    ):
        wid = lax.axis_index("s") * nc + lax.axis_index("c")

        @pl.when(wid < n_blocks)
        def _():
            pltpu.sync_copy(mask_hbm, mask_v)
            r0 = pl.multiple_of(wid * _SUB, _SUB)
            iota = lax.iota(jnp.int32, 16)
            row = lax.bitwise_and(iota, 7)
            half = lax.shift_right_logical(iota, 3)

            for c in range(_K // 16):
                lanes_v[pl.ds(16 * c, 16)] = lax.bitwise_and(
                    mask_v[pl.ds(16 * c, 16)], 127
                )

            def mask_scalar(j):
                pos = jnp.broadcast_to(j, (16,))
                return plsc.load_gather(mask_v, [pos])[0]

            def issue(j, carry):
                m = mask_scalar(j)
                mt = pl.multiple_of(
                    lax.shift_left(lax.shift_right_logical(m, 7), 7), _LANE
                )
                dst = pl.multiple_of(j * _SUB, _SUB)
                pltpu.async_copy(
                    x_hbm.at[pl.ds(r0, _SUB), pl.ds(mt, _LANE)],
                    blocks_v.at[pl.ds(dst, _SUB)],
                    sem,
                )
                return carry

            lax.fori_loop(0, _K, issue, 0)

            for q in range(4):
                pltpu.make_async_copy(
                    x_hbm.at[:, pl.ds(0, _LANE)],
                    blocks_v.at[pl.ds(q * _ROWS, _ROWS)],
                    sem,
                ).wait()

            def extract(p, carry):
                j0 = 2 * p
                col = half + j0
                lane = plsc.load_gather(lanes_v, [col])
                vec = plsc.load_gather(blocks_v, [col * _SUB + row, lane])
                plsc.store_scatter(vals_v, [row, col], vec)
                return carry

            lax.fori_loop(0, _K // 2, extract, 0)
            pltpu.sync_copy(vals_v, out_hbm.at[pl.ds(r0, _SUB), :])

    return gather_kernel


def kernel(x, mask):
    return _make_gather()(x, mask)
